# baseline (speedup 1.0000x reference)
import functools

import jax
import jax.numpy as jnp
from jax import lax
from jax.experimental import pallas as pl
from jax.experimental.pallas import tpu as pltpu
from jax.experimental.pallas import tpu_sc as plsc

_DIM = 768
_BATCH = 256
_SEQ = 512
_B = _BATCH * _SEQ
_NC = 2
_NS = 16
_NW = _NC * _NS
_BPW = _B // _NW
_CHUNK = 32
_NCHUNK = _BPW // _CHUNK
_NBUF = 4
_NGROUP = _NCHUNK // _NBUF

_mesh = plsc.VectorSubcoreMesh(core_axis_name="c", subcore_axis_name="s")


@functools.partial(
    pl.kernel,
    mesh=_mesh,
    out_type=jax.ShapeDtypeStruct((_B, _DIM), jnp.float32),
    scratch_types=[
        pltpu.VMEM((_NCHUNK, _CHUNK), jnp.int32),
    ] + [pltpu.VMEM((_CHUNK, _DIM), jnp.float32)] * _NBUF
      + [pltpu.SemaphoreType.DMA] * (2 * _NBUF),
)
def _emb_lookup(ids_hbm, table_hbm, out_hbm, idx_v,
                rows0, rows1, rows2, rows3,
                gsem0, gsem1, gsem2, gsem3,
                wsem0, wsem1, wsem2, wsem3):
    rows = (rows0, rows1, rows2, rows3)
    gsem = (gsem0, gsem1, gsem2, gsem3)
    wsem = (wsem0, wsem1, wsem2, wsem3)

    wid = lax.axis_index("s") * _NC + lax.axis_index("c")
    base = wid * _BPW

    pltpu.sync_copy(ids_hbm.at[wid], idx_v)

    def fire_gather(c, b):
        pltpu.async_copy(table_hbm.at[idx_v.at[c]], rows[b], gsem[b])

    def wait_gather(b):
        pltpu.make_async_copy(
            table_hbm.at[idx_v.at[0]], rows[b], gsem[b]).wait()

    def fire_write(c, b):
        pltpu.async_copy(
            rows[b], out_hbm.at[pl.ds(base + c * _CHUNK, _CHUNK)], wsem[b])

    def wait_write(b):
        pltpu.make_async_copy(
            rows[b], out_hbm.at[pl.ds(base, _CHUNK)], wsem[b]).wait()

    for b in range(_NBUF - 1):
        fire_gather(b, b)

    def group(g, _):
        for b in range(_NBUF):
            c = g * _NBUF + b
            pb = (b - 1) % _NBUF
            wait_gather(b)
            fire_write(c, b)
            if b == 0:
                @pl.when(g > 0)
                def _():
                    wait_write(pb)

                fire_gather(c + _NBUF - 1, pb)
            else:

                @pl.when(g + 1 < _NGROUP)
                def _():
                    wait_write(pb)
                    fire_gather(c + _NBUF - 1, pb)

        return ()

    lax.fori_loop(0, _NGROUP, group, ())

    for b in range(_NBUF):
        wait_write(b)


def kernel(table, input_ids):
    ids = input_ids.reshape(_NW, _NCHUNK, _CHUNK).astype(jnp.int32)
    out = _emb_lookup(ids, table)
    return out.reshape(_BATCH, _SEQ, _DIM)
